# baseline (speedup 1.0000x reference)
import functools

import jax
import jax.numpy as jnp
from jax import lax
from jax.experimental import pallas as pl
from jax.experimental.pallas import tpu as pltpu
from jax.experimental.pallas import tpu_sc as plsc

_N = 10000
_E = 320000
_D = 128

_NC = 2
_NS = 16
_NW = _NC * _NS
_EPW = _E // _NW
_K = 80
_CH = _EPW // _K

_HN = _N // 2
_AP = 5120
_APT = _AP // _NS
_TRASH = 5024

_mesh = plsc.VectorSubcoreMesh(core_axis_name="c", subcore_axis_name="s")


def _build_identity(idxb, base):
    iota = lax.iota(jnp.int32, 16)
    for r in range(_APT // _K):
        for v in range(_K // 16):
            idxb[r, pl.ds(v * 16, 16)] = base + r * _K + v * 16 + iota


def _localize(src_ref, dst_ref, off):
    @pl.loop(0, _CH)
    def _loop(j):
        for v in range(_K // 16):
            sl = pl.ds(v * 16, 16)
            d = src_ref[j, sl]
            ok = (d >= off) & (d < off + _HN)
            dst_ref[j, sl] = jnp.where(ok, d - off, _TRASH)


def _make_deg(off):
    @functools.partial(
        pl.kernel,
        out_type=(
            jax.ShapeDtypeStruct((_NC, _AP, _D), jnp.float32),
            jax.ShapeDtypeStruct((_NC, _AP, _D), jnp.float32),
        ),
        mesh=_mesh,
        scratch_types=[
            pltpu.VMEM((_CH, _K), jnp.int32),
            pltpu.VMEM((_CH, _K), jnp.int32),
            pltpu.VMEM((_CH, _K), jnp.int32),
            pltpu.VMEM((_K, _D), jnp.float32),
            pltpu.VMEM((_K, _D), jnp.float32),
            pltpu.VMEM((_APT // _K, _K), jnp.int32),
            pltpu.SemaphoreType.DMA,
            pltpu.VMEM_SHARED((_AP, _D), jnp.float32),
        ],
    )
    def _deg(src_hbm, dst_hbm, dsrc_out, ddst_out,
             sidx, didx, loc, ones_v, zbuf, idxb, ssem, acc):
        c = lax.axis_index("c")
        s = lax.axis_index("s")
        w = c * _NS + s
        base = pl.multiple_of(s * _APT, 8)

        pltpu.sync_copy(src_hbm.at[w], sidx)
        pltpu.sync_copy(dst_hbm.at[w], didx)

        one = jnp.ones((16,), jnp.float32)
        zero = jnp.zeros((16,), jnp.float32)

        @pl.loop(0, _K)
        def _fill_ones(i):
            for j in range(_D // 16):
                ones_v[i, pl.ds(j * 16, 16)] = one

        _build_identity(idxb, base)

        for idx_ref, out_ref in ((sidx, dsrc_out), (didx, ddst_out)):
            @pl.loop(0, _K)
            def _fill_zero(i):
                for j in range(_D // 16):
                    zbuf[i, pl.ds(j * 16, 16)] = zero

            _localize(idx_ref, loc, off)
            for r in range(_APT // _K):
                pltpu.sync_copy(zbuf, acc.at[idxb.at[r]])
            plsc.subcore_barrier()

            @pl.loop(0, _CH)
            def _scatter(j):
                pltpu.async_copy(ones_v, acc.at[loc.at[j]], ssem, add=True)

            @pl.loop(0, _CH)
            def _drain(j):
                pltpu.make_async_copy(ones_v, acc.at[loc.at[0]], ssem).wait()

            plsc.subcore_barrier()
            for r in range(_APT // _K):
                pltpu.sync_copy(acc.at[idxb.at[r]], zbuf)
                pltpu.sync_copy(zbuf, out_ref.at[c, pl.ds(base + r * _K, _K)])
            plsc.subcore_barrier()

    return _deg


def _make_spmm(off):
    @functools.partial(
        pl.kernel,
        out_type=jax.ShapeDtypeStruct((_NC, _AP, _D), jnp.float32),
        mesh=_mesh,
        scratch_types=[
            pltpu.VMEM((_CH, _K), jnp.int32),
            pltpu.VMEM((_CH, _K), jnp.int32),
            [pltpu.VMEM((_K, _D), jnp.float32)] * 4,
            pltpu.VMEM((_APT // _K, _K), jnp.int32),
            pltpu.SemaphoreType.DMA,
            pltpu.SemaphoreType.DMA,
            pltpu.VMEM_SHARED((_AP, _D), jnp.float32),
        ],
    )
    def _spmm(t_hbm, src_hbm, dst_hbm, p_out,
              sidx, didx, bufs, idxb, gsem, ssem, acc):
        c = lax.axis_index("c")
        s = lax.axis_index("s")
        w = c * _NS + s
        base = pl.multiple_of(s * _APT, 8)

        pltpu.sync_copy(src_hbm.at[w], sidx)
        pltpu.sync_copy(dst_hbm.at[w], didx)

        zero = jnp.zeros((16,), jnp.float32)

        @pl.loop(0, _K)
        def _fill_zero(i):
            for j in range(_D // 16):
                bufs[0][i, pl.ds(j * 16, 16)] = zero

        _localize(didx, didx, off)
        _build_identity(idxb, base)

        for r in range(_APT // _K):
            pltpu.sync_copy(bufs[0], acc.at[idxb.at[r]])
        plsc.subcore_barrier()

        def fire_gather(j, g):
            pltpu.async_copy(t_hbm.at[sidx.at[j]], bufs[g], gsem)

        def wait_gather(j, g):
            pltpu.make_async_copy(t_hbm.at[sidx.at[j]], bufs[g], gsem).wait()

        def fire_scatter(j, g):
            pltpu.async_copy(bufs[g], acc.at[didx.at[j]], ssem, add=True)

        def wait_scatter():
            pltpu.make_async_copy(bufs[0], acc.at[didx.at[0]], ssem).wait()

        fire_gather(0, 0)
        fire_gather(1, 1)
        wait_gather(0, 0)
        fire_scatter(0, 0)
        fire_gather(2, 2)
        wait_gather(1, 1)
        fire_scatter(1, 1)
        fire_gather(3, 3)

        @pl.loop(0, (_CH - 5) // 4)
        def _run(i):
            for g in range(4):
                j = 2 + 4 * i + g
                b = (2 + g) % 4
                wait_gather(j, b)
                fire_scatter(j, b)
                wait_scatter()
                fire_gather(j + 2, (b + 2) % 4)

        for j in range(_CH - 3, _CH):
            wait_gather(j, j % 4)
            fire_scatter(j, j % 4)
            if j + 2 < _CH:
                wait_scatter()
                fire_gather(j + 2, (j + 2) % 4)
        for _ in range(4):
            wait_scatter()

        plsc.subcore_barrier()
        for r in range(_APT // _K):
            pltpu.sync_copy(acc.at[idxb.at[r]], bufs[1])
            pltpu.sync_copy(bufs[1], p_out.at[c, pl.ds(base + r * _K, _K)])

    return _spmm


_deg_lo = _make_deg(0)
_deg_hi = _make_deg(_HN)
_spmm_lo = _make_spmm(0)
_spmm_hi = _make_spmm(_HN)


_R = 200
_GP = _HN // _R


def _pass_sel(lo_ref, hi_ref):
    r = pl.program_id(0)
    return jnp.where(r == 0, lo_ref[0] + lo_ref[1], hi_ref[0] + hi_ref[1])


def _norm_col(lo_ref, hi_ref):
    deg = _pass_sel(lo_ref, hi_ref)[:, 0:1]
    deg = jnp.where(deg > 0.0, deg, 1.0)
    return lax.rsqrt(deg)


def _tc1_body(feat_ref, dsl_ref, dsh_ref, w_ref, o_ref):
    ns = _norm_col(dsl_ref, dsh_ref)
    o_ref[...] = jnp.dot(feat_ref[...] * ns, w_ref[...],
                         preferred_element_type=jnp.float32)


def _tc2_body(plo_ref, phi_ref, dsl_ref, dsh_ref, ddl_ref, ddh_ref,
              b_ref, w_ref, o_ref):
    nd = _norm_col(ddl_ref, ddh_ref)
    ns = _norm_col(dsl_ref, dsh_ref)
    h = jax.nn.relu(_pass_sel(plo_ref, phi_ref) * nd + b_ref[...])
    o_ref[...] = jnp.dot(h * ns, w_ref[...],
                         preferred_element_type=jnp.float32)


def _tc3_body(plo_ref, phi_ref, ddl_ref, ddh_ref, b_ref, o_ref):
    nd = _norm_col(ddl_ref, ddh_ref)
    o_ref[...] = jax.nn.relu(_pass_sel(plo_ref, phi_ref) * nd + b_ref[...])


_lo_spec = pl.BlockSpec((_NC, _R, _D), lambda r, i: (0, i * (1 - r), 0))
_hi_spec = pl.BlockSpec((_NC, _R, _D), lambda r, i: (0, i * r, 0))
_row_spec = pl.BlockSpec((_R, _D), lambda r, i: (_GP * r + i, 0))
_b_spec = pl.BlockSpec((1, _D), lambda r, i: (0, 0))
_w_spec = pl.BlockSpec((_D, _D), lambda r, i: (0, 0))

_out_sds = jax.ShapeDtypeStruct((_N, _D), jnp.float32)

_tc1 = pl.pallas_call(
    _tc1_body,
    grid=(2, _GP),
    in_specs=[_row_spec, _lo_spec, _hi_spec, _w_spec],
    out_specs=_row_spec,
    out_shape=_out_sds,
)

_tc2 = pl.pallas_call(
    _tc2_body,
    grid=(2, _GP),
    in_specs=[_lo_spec, _hi_spec, _lo_spec, _hi_spec, _lo_spec, _hi_spec,
              _b_spec, _w_spec],
    out_specs=_row_spec,
    out_shape=_out_sds,
)

_tc3 = pl.pallas_call(
    _tc3_body,
    grid=(2, _GP),
    in_specs=[_lo_spec, _hi_spec, _lo_spec, _hi_spec, _b_spec],
    out_specs=_row_spec,
    out_shape=_out_sds,
)


def kernel(features, edge_index, W0, b0, W1, b1):
    src = edge_index[0].reshape(_NW, _CH, _K)
    dst = edge_index[1].reshape(_NW, _CH, _K)

    dsrc_lo, ddst_lo = _deg_lo(src, dst)
    dsrc_hi, ddst_hi = _deg_hi(src, dst)

    t1 = _tc1(features, dsrc_lo, dsrc_hi, W0)
    p1_lo = _spmm_lo(t1, src, dst)
    p1_hi = _spmm_hi(t1, src, dst)
    t2 = _tc2(p1_lo, p1_hi, dsrc_lo, dsrc_hi, ddst_lo, ddst_hi,
              b0.reshape(1, _D), W1)
    p2_lo = _spmm_lo(t2, src, dst)
    p2_hi = _spmm_hi(t2, src, dst)
    return _tc3(p2_lo, p2_hi, ddst_lo, ddst_hi, b1.reshape(1, _D))


# speedup vs baseline: 1.3655x; 1.3655x over previous
import functools

import jax
import jax.numpy as jnp
from jax import lax
from jax.experimental import pallas as pl
from jax.experimental.pallas import tpu as pltpu
from jax.experimental.pallas import tpu_sc as plsc

_N = 10000
_E = 320000
_D = 128

_NC = 2
_NS = 16
_NW = _NC * _NS
_EPW = _E // _NW
_K = 80
_CH = _EPW // _K

_HN = _N // 2
_AP = 5120
_APT = _AP // _NS
_TRASH = 5024

_mesh = plsc.VectorSubcoreMesh(core_axis_name="c", subcore_axis_name="s")


def _build_identity(idxb, base):
    iota = lax.iota(jnp.int32, 16)
    for r in range(_APT // _K):
        for v in range(_K // 16):
            idxb[r, pl.ds(v * 16, 16)] = base + r * _K + v * 16 + iota


def _localize(src_ref, dst_ref, off):
    @pl.loop(0, _CH)
    def _loop(j):
        for v in range(_K // 16):
            sl = pl.ds(v * 16, 16)
            d = src_ref[j, sl]
            ok = (d >= off) & (d < off + _HN)
            dst_ref[j, sl] = jnp.where(ok, d - off, _TRASH)


_NP = 10240
_NPT = _NP // _NS


@functools.partial(
    pl.kernel,
    out_type=(
        jax.ShapeDtypeStruct((_NC, _NP, _D), jnp.float32),
        jax.ShapeDtypeStruct((_NC, _NP, _D), jnp.float32),
    ),
    mesh=_mesh,
    scratch_types=[
        pltpu.VMEM((_CH, _K), jnp.int32),
        pltpu.VMEM((_K, _D), jnp.float32),
        pltpu.VMEM((_NPT // _K, _K), jnp.int32),
        pltpu.SemaphoreType.DMA,
        pltpu.VMEM_SHARED((_NP, _D), jnp.float32),
    ],
)
def _deg_full(src_hbm, dst_hbm, dsrc_out, ddst_out, idx, buf, idxb, ssem, acc):
    c = lax.axis_index("c")
    s = lax.axis_index("s")
    w = c * _NS + s
    base = pl.multiple_of(s * _NPT, 8)

    one = jnp.ones((16,), jnp.float32)
    zero = jnp.zeros((16,), jnp.float32)

    iota = lax.iota(jnp.int32, 16)
    for r in range(_NPT // _K):
        for v in range(_K // 16):
            idxb[r, pl.ds(v * 16, 16)] = base + r * _K + v * 16 + iota

    for in_hbm, out_ref in ((src_hbm, dsrc_out), (dst_hbm, ddst_out)):
        pltpu.sync_copy(in_hbm.at[w], idx)

        @pl.loop(0, _K)
        def _fill_zero(i):
            for j in range(_D // 16):
                buf[i, pl.ds(j * 16, 16)] = zero

        for r in range(_NPT // _K):
            pltpu.sync_copy(buf, acc.at[idxb.at[r]])
        plsc.subcore_barrier()

        @pl.loop(0, _K)
        def _fill_ones(i):
            for j in range(_D // 16):
                buf[i, pl.ds(j * 16, 16)] = one

        @pl.loop(0, _CH)
        def _scatter(j):
            pltpu.async_copy(buf, acc.at[idx.at[j]], ssem, add=True)

        @pl.loop(0, _CH)
        def _drain(j):
            pltpu.make_async_copy(buf, acc.at[idx.at[0]], ssem).wait()

        plsc.subcore_barrier()
        for r in range(_NPT // _K):
            pltpu.sync_copy(acc.at[idxb.at[r]], buf)
            pltpu.sync_copy(buf, out_ref.at[c, pl.ds(base + r * _K, _K)])
        plsc.subcore_barrier()


def _make_spmm(off):
    @functools.partial(
        pl.kernel,
        out_type=jax.ShapeDtypeStruct((_NC, _AP, _D), jnp.float32),
        mesh=_mesh,
        scratch_types=[
            pltpu.VMEM((_CH, _K), jnp.int32),
            pltpu.VMEM((_CH, _K), jnp.int32),
            [pltpu.VMEM((_K, _D), jnp.float32)] * 4,
            pltpu.VMEM((_APT // _K, _K), jnp.int32),
            pltpu.SemaphoreType.DMA,
            pltpu.SemaphoreType.DMA,
            pltpu.VMEM_SHARED((_AP, _D), jnp.float32),
        ],
    )
    def _spmm(t_hbm, src_hbm, dst_hbm, p_out,
              sidx, didx, bufs, idxb, gsem, ssem, acc):
        c = lax.axis_index("c")
        s = lax.axis_index("s")
        w = c * _NS + s
        base = pl.multiple_of(s * _APT, 8)

        pltpu.sync_copy(src_hbm.at[w], sidx)
        pltpu.sync_copy(dst_hbm.at[w], didx)

        zero = jnp.zeros((16,), jnp.float32)

        @pl.loop(0, _K)
        def _fill_zero(i):
            for j in range(_D // 16):
                bufs[0][i, pl.ds(j * 16, 16)] = zero

        _localize(didx, didx, off)
        _build_identity(idxb, base)

        for r in range(_APT // _K):
            pltpu.sync_copy(bufs[0], acc.at[idxb.at[r]])
        plsc.subcore_barrier()

        def fire_gather(j, g):
            pltpu.async_copy(t_hbm.at[sidx.at[j]], bufs[g], gsem)

        def wait_gather(j, g):
            pltpu.make_async_copy(t_hbm.at[sidx.at[j]], bufs[g], gsem).wait()

        def fire_scatter(j, g):
            pltpu.async_copy(bufs[g], acc.at[didx.at[j]], ssem, add=True)

        def wait_scatter():
            pltpu.make_async_copy(bufs[0], acc.at[didx.at[0]], ssem).wait()

        fire_gather(0, 0)
        fire_gather(1, 1)
        wait_gather(0, 0)
        fire_scatter(0, 0)
        fire_gather(2, 2)
        wait_gather(1, 1)
        fire_scatter(1, 1)
        fire_gather(3, 3)

        @pl.loop(0, (_CH - 5) // 4)
        def _run(i):
            for g in range(4):
                j = 2 + 4 * i + g
                b = (2 + g) % 4
                wait_gather(j, b)
                fire_scatter(j, b)
                wait_scatter()
                fire_gather(j + 2, (b + 2) % 4)

        for j in range(_CH - 3, _CH):
            wait_gather(j, j % 4)
            fire_scatter(j, j % 4)
            if j + 2 < _CH:
                wait_scatter()
                fire_gather(j + 2, (j + 2) % 4)
        for _ in range(4):
            wait_scatter()

        plsc.subcore_barrier()
        for r in range(_APT // _K):
            pltpu.sync_copy(acc.at[idxb.at[r]], bufs[1])
            pltpu.sync_copy(bufs[1], p_out.at[c, pl.ds(base + r * _K, _K)])

    return _spmm


_spmm_lo = _make_spmm(0)
_spmm_hi = _make_spmm(_HN)


_R = 200
_GP = _HN // _R


def _pass_sel(lo_ref, hi_ref):
    r = pl.program_id(0)
    return jnp.where(r == 0, lo_ref[0] + lo_ref[1], hi_ref[0] + hi_ref[1])


def _norm_col(d_ref):
    deg = (d_ref[0] + d_ref[1])[:, 0:1]
    deg = jnp.where(deg > 0.0, deg, 1.0)
    return lax.rsqrt(deg)


def _tc1_body(feat_ref, ds_ref, w_ref, o_ref):
    ns = _norm_col(ds_ref)
    o_ref[...] = jnp.dot(feat_ref[...] * ns, w_ref[...],
                         preferred_element_type=jnp.float32)


def _tc2_body(plo_ref, phi_ref, ds_ref, dd_ref, b_ref, w_ref, o_ref):
    nd = _norm_col(dd_ref)
    ns = _norm_col(ds_ref)
    h = jax.nn.relu(_pass_sel(plo_ref, phi_ref) * nd + b_ref[...])
    o_ref[...] = jnp.dot(h * ns, w_ref[...],
                         preferred_element_type=jnp.float32)


def _tc3_body(plo_ref, phi_ref, dd_ref, b_ref, o_ref):
    nd = _norm_col(dd_ref)
    o_ref[...] = jax.nn.relu(_pass_sel(plo_ref, phi_ref) * nd + b_ref[...])


_lo_spec = pl.BlockSpec((_NC, _R, _D), lambda r, i: (0, i * (1 - r), 0))
_hi_spec = pl.BlockSpec((_NC, _R, _D), lambda r, i: (0, i * r, 0))
_deg_spec = pl.BlockSpec((_NC, _R, _D), lambda r, i: (0, _GP * r + i, 0))
_row_spec = pl.BlockSpec((_R, _D), lambda r, i: (_GP * r + i, 0))
_b_spec = pl.BlockSpec((1, _D), lambda r, i: (0, 0))
_w_spec = pl.BlockSpec((_D, _D), lambda r, i: (0, 0))

_out_sds = jax.ShapeDtypeStruct((_N, _D), jnp.float32)

_tc1 = pl.pallas_call(
    _tc1_body,
    grid=(2, _GP),
    in_specs=[_row_spec, _deg_spec, _w_spec],
    out_specs=_row_spec,
    out_shape=_out_sds,
)

_tc2 = pl.pallas_call(
    _tc2_body,
    grid=(2, _GP),
    in_specs=[_lo_spec, _hi_spec, _deg_spec, _deg_spec,
              _b_spec, _w_spec],
    out_specs=_row_spec,
    out_shape=_out_sds,
)

_tc3 = pl.pallas_call(
    _tc3_body,
    grid=(2, _GP),
    in_specs=[_lo_spec, _hi_spec, _deg_spec, _b_spec],
    out_specs=_row_spec,
    out_shape=_out_sds,
)


def kernel(features, edge_index, W0, b0, W1, b1):
    src = edge_index[0].reshape(_NW, _CH, _K)
    dst = edge_index[1].reshape(_NW, _CH, _K)

    dsrc, ddst = _deg_full(src, dst)

    t1 = _tc1(features, dsrc, W0)
    p1_lo = _spmm_lo(t1, src, dst)
    p1_hi = _spmm_hi(t1, src, dst)
    t2 = _tc2(p1_lo, p1_hi, dsrc, ddst, b0.reshape(1, _D), W1)
    p2_lo = _spmm_lo(t2, src, dst)
    p2_hi = _spmm_hi(t2, src, dst)
    return _tc3(p2_lo, p2_hi, ddst, b1.reshape(1, _D))


# speedup vs baseline: 1.8408x; 1.3481x over previous
import functools

import jax
import jax.numpy as jnp
from jax import lax
from jax.experimental import pallas as pl
from jax.experimental.pallas import tpu as pltpu
from jax.experimental.pallas import tpu_sc as plsc

_N = 10000
_E = 320000
_D = 128

_NC = 2
_NS = 16
_NW = _NC * _NS
_EPW = _E // _NW
_K = 80
_CH = _EPW // _K

_HN = _N // 2
_AP = 5120
_APT = _AP // _NS
_TRASH = 5024

_mesh = plsc.VectorSubcoreMesh(core_axis_name="c", subcore_axis_name="s")


def _build_identity(idxb, base):
    iota = lax.iota(jnp.int32, 16)
    for r in range(_APT // _K):
        for v in range(_K // 16):
            idxb[r, pl.ds(v * 16, 16)] = base + r * _K + v * 16 + iota


def _localize(src_ref, dst_ref, off):
    @pl.loop(0, _CH)
    def _loop(j):
        for v in range(_K // 16):
            sl = pl.ds(v * 16, 16)
            d = src_ref[j, sl]
            ok = (d >= off) & (d < off + _HN)
            dst_ref[j, sl] = jnp.where(ok, d - off, _TRASH)


_NP = 10240
_NPT = _NP // _NS


@functools.partial(
    pl.kernel,
    out_type=(
        jax.ShapeDtypeStruct((_NC, _NP, _D), jnp.float32),
        jax.ShapeDtypeStruct((_NC, _NP, _D), jnp.float32),
    ),
    mesh=_mesh,
    scratch_types=[
        pltpu.VMEM((_CH, _K), jnp.int32),
        pltpu.VMEM((_K, _D), jnp.float32),
        pltpu.VMEM((_NPT // _K, _K), jnp.int32),
        pltpu.SemaphoreType.DMA,
        pltpu.VMEM_SHARED((_NP, _D), jnp.float32),
    ],
)
def _deg_full(src_hbm, dst_hbm, dsrc_out, ddst_out, idx, buf, idxb, ssem, acc):
    c = lax.axis_index("c")
    s = lax.axis_index("s")
    w = c * _NS + s
    base = pl.multiple_of(s * _NPT, 8)

    one = jnp.ones((16,), jnp.float32)
    zero = jnp.zeros((16,), jnp.float32)

    iota = lax.iota(jnp.int32, 16)
    for r in range(_NPT // _K):
        for v in range(_K // 16):
            idxb[r, pl.ds(v * 16, 16)] = base + r * _K + v * 16 + iota

    for in_hbm, out_ref in ((src_hbm, dsrc_out), (dst_hbm, ddst_out)):
        pltpu.sync_copy(in_hbm.at[w], idx)

        @pl.loop(0, _K)
        def _fill_zero(i):
            for j in range(_D // 16):
                buf[i, pl.ds(j * 16, 16)] = zero

        for r in range(_NPT // _K):
            pltpu.sync_copy(buf, acc.at[idxb.at[r]])
        plsc.subcore_barrier()

        @pl.loop(0, _K)
        def _fill_ones(i):
            for j in range(_D // 16):
                buf[i, pl.ds(j * 16, 16)] = one

        @pl.loop(0, _CH)
        def _scatter(j):
            pltpu.async_copy(buf, acc.at[idx.at[j]], ssem, add=True)

        @pl.loop(0, _CH)
        def _drain(j):
            pltpu.make_async_copy(buf, acc.at[idx.at[0]], ssem).wait()

        plsc.subcore_barrier()
        for r in range(_NPT // _K):
            pltpu.sync_copy(acc.at[idxb.at[r]], buf)
            pltpu.sync_copy(buf, out_ref.at[c, pl.ds(base + r * _K, _K)])
        plsc.subcore_barrier()


@functools.partial(
    pl.kernel,
    out_type=jax.ShapeDtypeStruct((_NC, _NP, _D), jnp.float32),
    mesh=_mesh,
    scratch_types=[
        pltpu.VMEM((64, _K), jnp.int32),
        pltpu.VMEM((64, _K), jnp.int32),
        [pltpu.VMEM((_K, _D), jnp.float32)] * 2,
        pltpu.VMEM((_NPT // _K, _K), jnp.int32),
        pltpu.SemaphoreType.DMA,
        pltpu.SemaphoreType.DMA,
        pltpu.VMEM_SHARED((_NP, _D), jnp.float32),
    ],
)
def _spmm_full(t_hbm, src_hbm, dst_hbm, p_out,
               sidx, didx, bufs, idxb, gsem, ssem, acc):
    c = lax.axis_index("c")
    s = lax.axis_index("s")
    w = c * _NS + s
    base = pl.multiple_of(s * _NPT, 8)

    zero = jnp.zeros((16,), jnp.float32)

    @pl.loop(0, _K)
    def _fill_zero(i):
        for j in range(_D // 16):
            bufs[0][i, pl.ds(j * 16, 16)] = zero

    iota = lax.iota(jnp.int32, 16)
    for r in range(_NPT // _K):
        for v in range(_K // 16):
            idxb[r, pl.ds(v * 16, 16)] = base + r * _K + v * 16 + iota

    for r in range(_NPT // _K):
        pltpu.sync_copy(bufs[0], acc.at[idxb.at[r]])
    plsc.subcore_barrier()

    def fire_gather(j, g):
        pltpu.async_copy(t_hbm.at[sidx.at[j]], bufs[g], gsem)

    def wait_gather(j, g):
        pltpu.make_async_copy(t_hbm.at[sidx.at[j]], bufs[g], gsem).wait()

    def fire_scatter(j, g):
        pltpu.async_copy(bufs[g], acc.at[didx.at[j]], ssem, add=True)

    def wait_scatter():
        pltpu.make_async_copy(bufs[0], acc.at[didx.at[0]], ssem).wait()

    def _ring(nch):
        fire_gather(0, 0)
        wait_gather(0, 0)
        fire_scatter(0, 0)
        fire_gather(1, 1)
        m = (nch - 3) // 2

        @pl.loop(0, m)
        def _run(i):
            for g in range(2):
                j = 1 + 2 * i + g
                b = (1 + g) % 2
                wait_gather(j, b)
                fire_scatter(j, b)
                wait_scatter()
                fire_gather(j + 1, g)

        for j in range(1 + 2 * m, nch):
            wait_gather(j, j % 2)
            fire_scatter(j, j % 2)
            if j + 1 < nch:
                wait_scatter()
                fire_gather(j + 1, (j + 1) % 2)
        for _ in range(2):
            wait_scatter()

    pltpu.sync_copy(src_hbm.at[w, pl.ds(0, 64)], sidx)
    pltpu.sync_copy(dst_hbm.at[w, pl.ds(0, 64)], didx)
    _ring(64)
    pltpu.sync_copy(src_hbm.at[w, pl.ds(64, 61)], sidx.at[pl.ds(0, 61)])
    pltpu.sync_copy(dst_hbm.at[w, pl.ds(64, 61)], didx.at[pl.ds(0, 61)])
    _ring(61)

    plsc.subcore_barrier()
    for r in range(_NPT // _K):
        pltpu.sync_copy(acc.at[idxb.at[r]], bufs[1])
        pltpu.sync_copy(bufs[1], p_out.at[c, pl.ds(base + r * _K, _K)])


_R = 200
_GP = _HN // _R


def _norm_col(d_ref):
    deg = (d_ref[0] + d_ref[1])[:, 0:1]
    deg = jnp.where(deg > 0.0, deg, 1.0)
    return lax.rsqrt(deg)


def _tc1_body(feat_ref, ds_ref, w_ref, o_ref):
    ns = _norm_col(ds_ref)
    o_ref[...] = jnp.dot(feat_ref[...] * ns, w_ref[...],
                         preferred_element_type=jnp.float32)


def _tc2_body(p_ref, ds_ref, dd_ref, b_ref, w_ref, o_ref):
    nd = _norm_col(dd_ref)
    ns = _norm_col(ds_ref)
    h = jax.nn.relu((p_ref[0] + p_ref[1]) * nd + b_ref[...])
    o_ref[...] = jnp.dot(h * ns, w_ref[...],
                         preferred_element_type=jnp.float32)


def _tc3_body(p_ref, dd_ref, b_ref, o_ref):
    nd = _norm_col(dd_ref)
    o_ref[...] = jax.nn.relu((p_ref[0] + p_ref[1]) * nd + b_ref[...])


_deg_spec = pl.BlockSpec((_NC, _R, _D), lambda r, i: (0, _GP * r + i, 0))
_row_spec = pl.BlockSpec((_R, _D), lambda r, i: (_GP * r + i, 0))
_b_spec = pl.BlockSpec((1, _D), lambda r, i: (0, 0))
_w_spec = pl.BlockSpec((_D, _D), lambda r, i: (0, 0))

_out_sds = jax.ShapeDtypeStruct((_N, _D), jnp.float32)

_tc1 = pl.pallas_call(
    _tc1_body,
    grid=(2, _GP),
    in_specs=[_row_spec, _deg_spec, _w_spec],
    out_specs=_row_spec,
    out_shape=_out_sds,
)

_tc2 = pl.pallas_call(
    _tc2_body,
    grid=(2, _GP),
    in_specs=[_deg_spec, _deg_spec, _deg_spec, _b_spec, _w_spec],
    out_specs=_row_spec,
    out_shape=_out_sds,
)

_tc3 = pl.pallas_call(
    _tc3_body,
    grid=(2, _GP),
    in_specs=[_deg_spec, _deg_spec, _b_spec],
    out_specs=_row_spec,
    out_shape=_out_sds,
)


def kernel(features, edge_index, W0, b0, W1, b1):
    src = edge_index[0].reshape(_NW, _CH, _K)
    dst = edge_index[1].reshape(_NW, _CH, _K)

    dsrc, ddst = _deg_full(src, dst)

    t1 = _tc1(features, dsrc, W0)
    p1 = _spmm_full(t1, src, dst)
    t2 = _tc2(p1, dsrc, ddst, b0.reshape(1, _D), W1)
    p2 = _spmm_full(t2, src, dst)
    return _tc3(p2, ddst, b1.reshape(1, _D))
